# baseline (speedup 1.0000x reference)
import jax
import jax.numpy as jnp
from jax.experimental import pallas as pl
from jax.experimental.pallas import tpu as pltpu

_BI = 400


def _gcn_body(x_ref, adj_ref, w1_ref, b1_ref, w2_ref, b2_ref,
              out_ref, h_ref, s1_ref, s2_ref):
    phase = pl.program_id(0)
    i = pl.program_id(1)

    @pl.when(jnp.logical_and(phase == 0, i == 0))
    def _():
        s1_ref[...] = jnp.dot(x_ref[...], w1_ref[...],
                              preferred_element_type=jnp.float32
                              ).astype(jnp.bfloat16)

    adj_bf = adj_ref[...].astype(jnp.bfloat16)

    @pl.when(phase == 0)
    def _():
        acc = jnp.dot(adj_bf, s1_ref[...],
                      preferred_element_type=jnp.float32)
        h = jnp.maximum(acc + b1_ref[...], 0.0)
        h_ref[...] = h
        s2_ref[pl.ds(i * _BI, _BI), :] = jnp.dot(
            h.astype(jnp.bfloat16), w2_ref[...].astype(jnp.bfloat16),
            preferred_element_type=jnp.float32).astype(jnp.bfloat16)

    @pl.when(phase == 1)
    def _():
        logits = jnp.dot(adj_bf, s2_ref[...],
                         preferred_element_type=jnp.float32) + b2_ref[...]
        m = jnp.max(logits, axis=-1, keepdims=True)
        lse = jnp.log(jnp.sum(jnp.exp(logits - m), axis=-1, keepdims=True)) + m
        out_ref[...] = logits - lse


def kernel(x, adj, W1, bias1, W2, bias2):
    n, nfeat = x.shape
    nhid = W1.shape[1]
    ncls = W2.shape[1]
    b1 = bias1.reshape(1, nhid)
    b2 = bias2.reshape(1, ncls)

    out, h = pl.pallas_call(
        _gcn_body,
        grid=(2, n // _BI),
        in_specs=[
            pl.BlockSpec((n, nfeat), lambda p, i: (0, 0)),
            pl.BlockSpec((_BI, n), lambda p, i: (i, 0)),
            pl.BlockSpec((nfeat, nhid), lambda p, i: (0, 0)),
            pl.BlockSpec((1, nhid), lambda p, i: (0, 0)),
            pl.BlockSpec((nhid, ncls), lambda p, i: (0, 0)),
            pl.BlockSpec((1, ncls), lambda p, i: (0, 0)),
        ],
        out_specs=[
            pl.BlockSpec((_BI, ncls), lambda p, i: (p * i, 0)),
            pl.BlockSpec((_BI, nhid),
                         lambda p, i: (i + p * (n // _BI - 1 - i), 0)),
        ],
        out_shape=[
            jax.ShapeDtypeStruct((n, ncls), jnp.float32),
            jax.ShapeDtypeStruct((n, nhid), jnp.float32),
        ],
        scratch_shapes=[
            pltpu.VMEM((n, nhid), jnp.bfloat16),
            pltpu.VMEM((n, ncls), jnp.bfloat16),
        ],
        compiler_params=pltpu.CompilerParams(
            dimension_semantics=("arbitrary", "arbitrary"),
        ),
    )(x, adj, W1, b1, W2, b2)

    return (out, h)


# speedup vs baseline: 1.0194x; 1.0194x over previous
import jax
import jax.numpy as jnp
from jax.experimental import pallas as pl
from jax.experimental.pallas import tpu as pltpu

_BI = 400


def _gcn_body(x_ref, adj_ref, w1_ref, b1_ref, w2_ref, b2_ref,
              out_ref, h_ref, s1_ref, s2_ref):
    phase = pl.program_id(0)
    i = pl.program_id(1)

    @pl.when(jnp.logical_and(phase == 0, i == 0))
    def _():
        s1_ref[...] = jnp.dot(x_ref[...], w1_ref[...],
                              preferred_element_type=jnp.float32)

    @pl.when(phase == 0)
    def _():
        acc = jnp.dot(adj_ref[...], s1_ref[...],
                      preferred_element_type=jnp.float32)
        h = jnp.maximum(acc + b1_ref[...], 0.0)
        h_ref[...] = h
        s2_ref[pl.ds(i * _BI, _BI), :] = jnp.dot(
            h, w2_ref[...], preferred_element_type=jnp.float32)

    @pl.when(phase == 1)
    def _():
        logits = jnp.dot(adj_ref[...], s2_ref[...],
                         preferred_element_type=jnp.float32) + b2_ref[...]
        m = jnp.max(logits, axis=-1, keepdims=True)
        lse = jnp.log(jnp.sum(jnp.exp(logits - m), axis=-1, keepdims=True)) + m
        out_ref[...] = logits - lse


def kernel(x, adj, W1, bias1, W2, bias2):
    n, nfeat = x.shape
    nhid = W1.shape[1]
    ncls = W2.shape[1]
    b1 = bias1.reshape(1, nhid)
    b2 = bias2.reshape(1, ncls)

    out, h = pl.pallas_call(
        _gcn_body,
        grid=(2, n // _BI),
        in_specs=[
            pl.BlockSpec((n, nfeat), lambda p, i: (0, 0)),
            pl.BlockSpec((_BI, n), lambda p, i: (i, 0)),
            pl.BlockSpec((nfeat, nhid), lambda p, i: (0, 0)),
            pl.BlockSpec((1, nhid), lambda p, i: (0, 0)),
            pl.BlockSpec((nhid, ncls), lambda p, i: (0, 0)),
            pl.BlockSpec((1, ncls), lambda p, i: (0, 0)),
        ],
        out_specs=[
            pl.BlockSpec((_BI, ncls), lambda p, i: (p * i, 0)),
            pl.BlockSpec((_BI, nhid),
                         lambda p, i: (i + p * (n // _BI - 1 - i), 0)),
        ],
        out_shape=[
            jax.ShapeDtypeStruct((n, ncls), jnp.float32),
            jax.ShapeDtypeStruct((n, nhid), jnp.float32),
        ],
        scratch_shapes=[
            pltpu.VMEM((n, nhid), jnp.float32),
            pltpu.VMEM((n, ncls), jnp.float32),
        ],
        compiler_params=pltpu.CompilerParams(
            dimension_semantics=("arbitrary", "arbitrary"),
        ),
    )(x, adj, W1, b1, W2, b2)

    return (out, h)


# speedup vs baseline: 1.0206x; 1.0012x over previous
import jax
import jax.numpy as jnp
from jax.experimental import pallas as pl
from jax.experimental.pallas import tpu as pltpu

_BI = 400


def _gcn_body(x_ref, adj_ref, w1_ref, b1_ref, w2_ref, b2_ref,
              out_ref, h_ref, s1_ref, s2_ref):
    phase = pl.program_id(0)
    i = pl.program_id(1)

    @pl.when(jnp.logical_and(phase == 0, i == 0))
    def _():
        s1_ref[...] = jnp.dot(x_ref[...], w1_ref[...],
                              preferred_element_type=jnp.float32)

    @pl.when(phase == 0)
    def _():
        acc = jnp.dot(adj_ref[...], s1_ref[...],
                      preferred_element_type=jnp.float32)
        h = jnp.maximum(acc + b1_ref[...], 0.0)
        h_ref[...] = h
        s2_ref[pl.ds(i * _BI, _BI), :] = jnp.dot(
            h, w2_ref[...], preferred_element_type=jnp.float32)

    @pl.when(phase == 1)
    def _():
        logits = jnp.dot(adj_ref[...], s2_ref[...],
                         preferred_element_type=jnp.float32) + b2_ref[...]
        m = jnp.max(logits, axis=-1, keepdims=True)
        lse = jnp.log(jnp.sum(jnp.exp(logits - m), axis=-1, keepdims=True)) + m
        out_ref[...] = logits - lse


def kernel(x, adj, W1, bias1, W2, bias2):
    n, nfeat = x.shape
    nhid = W1.shape[1]
    ncls = W2.shape[1]
    b1 = bias1.reshape(1, nhid)
    b2 = bias2.reshape(1, ncls)

    nb = n // _BI
    out, h = pl.pallas_call(
        _gcn_body,
        grid=(2, nb),
        in_specs=[
            pl.BlockSpec((n, nfeat), lambda p, i: (0, 0)),
            pl.BlockSpec((_BI, n),
                         lambda p, i: (i + p * (nb - 1 - 2 * i), 0)),
            pl.BlockSpec((nfeat, nhid), lambda p, i: (0, 0)),
            pl.BlockSpec((1, nhid), lambda p, i: (0, 0)),
            pl.BlockSpec((nhid, ncls), lambda p, i: (0, 0)),
            pl.BlockSpec((1, ncls), lambda p, i: (0, 0)),
        ],
        out_specs=[
            pl.BlockSpec((_BI, ncls), lambda p, i: (nb - 1 - p * i, 0)),
            pl.BlockSpec((_BI, nhid),
                         lambda p, i: (i + p * (nb - 1 - i), 0)),
        ],
        out_shape=[
            jax.ShapeDtypeStruct((n, ncls), jnp.float32),
            jax.ShapeDtypeStruct((n, nhid), jnp.float32),
        ],
        scratch_shapes=[
            pltpu.VMEM((n, nhid), jnp.float32),
            pltpu.VMEM((n, ncls), jnp.float32),
        ],
        compiler_params=pltpu.CompilerParams(
            dimension_semantics=("arbitrary", "arbitrary"),
        ),
    )(x, adj, W1, b1, W2, b2)

    return (out, h)
